# baseline (speedup 1.0000x reference)
import functools

import jax
import jax.numpy as jnp
import numpy as np
from jax import lax
from jax.experimental import pallas as pl
from jax.experimental.pallas import tpu as pltpu
from jax.experimental.pallas import tpu_sc as plsc

N = 10000
E = 320000
D = 128

NC = 2
NS = 16
NW = NC * NS

CHUNK = 120
CPW = 84
EPW = CPW * CHUNK
E_PAD = EPW * NW

ROWS_PER_TILE = 632
N_SP = ROWS_PER_TILE * NS

_PERM = np.empty((D,), np.int32)
for _g in range(D // 32):
    for _j in range(16):
        _PERM[32 * _g + _j] = 32 * _g + 2 * _j
        _PERM[32 * _g + 16 + _j] = 32 * _g + 2 * _j + 1

_sc_mesh = plsc.VectorSubcoreMesh(core_axis_name="c", subcore_axis_name="s")


def _widen_rows(wi, f32):
    def rows4(i, carry):
        for m in range(4):
            row = i * 4 + m
            for k in range(D // 32):
                w = wi[row, pl.ds(16 * k, 16)]
                a = lax.bitcast_convert_type(jnp.left_shift(w, 16),
                                             jnp.float32)
                b = lax.bitcast_convert_type(jnp.bitwise_and(w, -65536),
                                             jnp.float32)
                f32[row, pl.ds(32 * k, 16)] = a
                f32[row, pl.ds(32 * k + 16, 16)] = b
        return carry

    lax.fori_loop(0, CHUNK // 4, rows4, 0)


@functools.partial(
    pl.kernel,
    out_type=jax.ShapeDtypeStruct((NC, N_SP, D), jnp.float32),
    mesh=_sc_mesh,
    compiler_params=pltpu.CompilerParams(use_tc_tiling_on_sc=False),
    scratch_types=[
        pltpu.VMEM((3, CHUNK), jnp.int32),
        pltpu.VMEM((4, CHUNK), jnp.int32),
        pltpu.VMEM((2, CHUNK, D // 2), jnp.int32),
        pltpu.VMEM((2, CHUNK, D), jnp.float32),
        pltpu.VMEM_SHARED((N_SP, D), jnp.float32),
        [pltpu.SemaphoreType.DMA] * 2,
        [pltpu.SemaphoreType.DMA] * 2,
        [pltpu.SemaphoreType.DMA] * 3,
        [pltpu.SemaphoreType.DMA] * 4,
        pltpu.SemaphoreType.DMA,
    ],
)
def _sc_agg(h_hbm, src_hbm, dst_hbm, zeros_hbm, out_hbm,
            src_v, dst_v, rowsb, rows32, agg_sh, gsems, ssems, isrc, idst,
            zsem):
    cid = lax.axis_index("c")
    sid = lax.axis_index("s")
    wid = cid * NS + sid

    row0 = sid * ROWS_PER_TILE
    pltpu.async_copy(zeros_hbm.at[pl.ds(row0, ROWS_PER_TILE)],
                     agg_sh.at[pl.ds(row0, ROWS_PER_TILE)], zsem)
    for s in (0, 1, 2):
        pltpu.async_copy(src_hbm.at[wid, s], src_v.at[s], isrc[s])
        pltpu.async_copy(dst_hbm.at[wid, s], dst_v.at[s], idst[s])
    pltpu.make_async_copy(src_hbm.at[wid, 0], src_v.at[0], isrc[0]).wait()
    pltpu.async_copy(h_hbm.at[src_v.at[0]], rowsb.at[0], gsems[0])
    pltpu.make_async_copy(src_hbm.at[wid, 1], src_v.at[1], isrc[1]).wait()
    pltpu.async_copy(h_hbm.at[src_v.at[1]], rowsb.at[1], gsems[1])
    pltpu.make_async_copy(zeros_hbm.at[pl.ds(row0, ROWS_PER_TILE)],
                          agg_sh.at[pl.ds(row0, ROWS_PER_TILE)], zsem).wait()
    plsc.subcore_barrier()

    def _step(j, u):
        p, r3, d = u % 2, u % 3, u % 4
        q, r2, d1 = 1 - (u % 2), (u + 2) % 3, (u + 3) % 4

        pltpu.make_async_copy(h_hbm.at[src_v.at[r3]], rowsb.at[p],
                              gsems[p]).wait()

        @pl.when(j >= 1)
        def _():
            pltpu.make_async_copy(rows32.at[q], agg_sh.at[dst_v.at[d1]],
                                  ssems[q]).wait()

        _widen_rows(rowsb.at[p], rows32.at[p])

        pltpu.make_async_copy(dst_hbm.at[wid, j], dst_v.at[d],
                              idst[d]).wait()

        pltpu.async_copy(rows32.at[p], agg_sh.at[dst_v.at[d]], ssems[p],
                         add=True)

        @pl.when(j + 3 < CPW)
        def _():
            pltpu.async_copy(src_hbm.at[wid, j + 3], src_v.at[r3], isrc[r3])
            pltpu.async_copy(dst_hbm.at[wid, j + 3], dst_v.at[d1], idst[d1])

        @pl.when(j + 2 < CPW)
        def _():
            pltpu.make_async_copy(src_hbm.at[wid, j + 2], src_v.at[r2],
                                  isrc[r2]).wait()
            pltpu.async_copy(h_hbm.at[src_v.at[r2]], rowsb.at[p], gsems[p])

    def body(i, carry):
        for u in range(12):
            _step(12 * i + u, u)
        return carry

    lax.fori_loop(0, CPW // 12, body, 0)
    pltpu.make_async_copy(rows32.at[(CPW - 1) % 2],
                          agg_sh.at[dst_v.at[(CPW - 1) % 4]],
                          ssems[(CPW - 1) % 2]).wait()
    plsc.subcore_barrier()

    pltpu.sync_copy(agg_sh.at[pl.ds(row0, ROWS_PER_TILE)],
                    out_hbm.at[cid, pl.ds(row0, ROWS_PER_TILE)])


ROW_BLK = 2000


def _tc_mlp_body(h_ref, a0_ref, a1_ref, w1_ref, w1p_ref, b1_ref, w2_ref,
                 b2_ref, out_ref, outb_ref):
    ap = a0_ref[0] + a1_ref[0]
    t = jnp.dot(h_ref[...], w1_ref[...], preferred_element_type=jnp.float32)
    t = t + jnp.dot(ap, w1p_ref[...], preferred_element_type=jnp.float32)
    t = jnp.maximum(t + b1_ref[...], 0.0)
    u = jnp.dot(t, w2_ref[...], preferred_element_type=jnp.float32) + b2_ref[...]
    o = jnp.where(u > 0.0, u, jnp.exp(jnp.minimum(u, 0.0)) - 1.0)
    out_ref[...] = o
    outb_ref[...] = o.astype(jnp.bfloat16)


def _tc_mlp(h, agg, w1, w1p, b1, w2, b2):
    grid = N // ROW_BLK
    return pl.pallas_call(
        _tc_mlp_body,
        grid=(grid,),
        in_specs=[
            pl.BlockSpec((ROW_BLK, D), lambda i: (i, 0)),
            pl.BlockSpec((1, ROW_BLK, D), lambda i: (0, i, 0)),
            pl.BlockSpec((1, ROW_BLK, D), lambda i: (1, i, 0)),
            pl.BlockSpec((D, D), lambda i: (0, 0)),
            pl.BlockSpec((D, D), lambda i: (0, 0)),
            pl.BlockSpec((1, D), lambda i: (0, 0)),
            pl.BlockSpec((D, D), lambda i: (0, 0)),
            pl.BlockSpec((1, D), lambda i: (0, 0)),
        ],
        out_specs=[
            pl.BlockSpec((ROW_BLK, D), lambda i: (i, 0)),
            pl.BlockSpec((ROW_BLK, D), lambda i: (i, 0)),
        ],
        out_shape=[
            jax.ShapeDtypeStruct((N, D), jnp.float32),
            jax.ShapeDtypeStruct((N, D), jnp.bfloat16),
        ],
    )(h, agg, agg, w1, w1p, b1, w2, b2)


def kernel(x, edge_index, W1_0, b1_0, W2_0, b2_0, W1_1, b1_1, W2_1, b2_1,
           W1_2, b1_2, W2_2, b2_2):
    src = edge_index[0]
    dst = edge_index[1]
    pad = E_PAD - E
    pad_iota = np.arange(pad, dtype=np.int32)
    pad_src = jnp.asarray(pad_iota % N)
    pad_dst = jnp.asarray(N + pad_iota % (N_SP - N))
    src_p = jnp.concatenate([src, pad_src]).reshape(NW, CPW, CHUNK)
    dst_p = jnp.concatenate([dst, pad_dst]).reshape(NW, CPW, CHUNK)
    zeros = jnp.zeros((N_SP, D), jnp.float32)

    params = [(W1_0, b1_0, W2_0, b2_0), (W1_1, b1_1, W2_1, b2_1),
              (W1_2, b1_2, W2_2, b2_2)]
    h = x
    hb = x.astype(jnp.bfloat16)
    for (w1, b1, w2, b2) in params:
        hb32 = lax.bitcast_convert_type(hb.reshape(N, D // 2, 2), jnp.int32)
        agg = _sc_agg(hb32, src_p, dst_p, zeros)
        h, hb = _tc_mlp(h, agg, w1, w1[_PERM, :], b1.reshape(1, D),
                        w2, b2.reshape(1, D))
    return h


# speedup vs baseline: 2.4906x; 2.4906x over previous
import functools

import jax
import jax.numpy as jnp
import numpy as np
from jax import lax
from jax.experimental import pallas as pl
from jax.experimental.pallas import tpu as pltpu
from jax.experimental.pallas import tpu_sc as plsc

N = 10000
E = 320000
D = 128

NC = 2
NS = 16
NW = NC * NS

CHUNK = 120
CPW = 84
EPW = CPW * CHUNK
E_PAD = EPW * NW

ROWS_PER_TILE = 632
N_SP = ROWS_PER_TILE * NS

_sc_mesh = plsc.VectorSubcoreMesh(core_axis_name="c", subcore_axis_name="s")


@functools.partial(
    pl.kernel,
    out_type=jax.ShapeDtypeStruct((NC, N_SP, D), jnp.float32),
    mesh=_sc_mesh,
    scratch_types=[
        pltpu.VMEM((3, CHUNK), jnp.int32),
        pltpu.VMEM((4, CHUNK), jnp.int32),
        pltpu.VMEM((3, CHUNK, D), jnp.float32),
        pltpu.VMEM_SHARED((N_SP, D), jnp.float32),
        [pltpu.SemaphoreType.DMA] * 3,
        [pltpu.SemaphoreType.DMA] * 2,
        [pltpu.SemaphoreType.DMA] * 3,
        [pltpu.SemaphoreType.DMA] * 4,
        pltpu.SemaphoreType.DMA,
    ],
)
def _sc_agg(h_hbm, src_hbm, dst_hbm, zeros_hbm, out_hbm,
            src_v, dst_v, rows_v, agg_sh, gsems, ssems, isrc, idst, zsem):
    cid = lax.axis_index("c")
    sid = lax.axis_index("s")
    wid = cid * NS + sid

    row0 = sid * ROWS_PER_TILE
    pltpu.async_copy(zeros_hbm.at[pl.ds(row0, ROWS_PER_TILE)],
                     agg_sh.at[pl.ds(row0, ROWS_PER_TILE)], zsem)
    for s in (0, 1, 2):
        pltpu.async_copy(src_hbm.at[wid, s], src_v.at[s], isrc[s])
        pltpu.async_copy(dst_hbm.at[wid, s], dst_v.at[s], idst[s])
    pltpu.make_async_copy(src_hbm.at[wid, 0], src_v.at[0], isrc[0]).wait()
    pltpu.async_copy(h_hbm.at[src_v.at[0]], rows_v.at[0], gsems[0])
    pltpu.make_async_copy(src_hbm.at[wid, 1], src_v.at[1], isrc[1]).wait()
    pltpu.async_copy(h_hbm.at[src_v.at[1]], rows_v.at[1], gsems[1])
    pltpu.make_async_copy(zeros_hbm.at[pl.ds(row0, ROWS_PER_TILE)],
                          agg_sh.at[pl.ds(row0, ROWS_PER_TILE)], zsem).wait()
    plsc.subcore_barrier()

    def _step(j, u):
        r, d, p = u % 3, u % 4, u % 2
        r1, d1, p1 = (u + 2) % 3, (u + 3) % 4, 1 - (u % 2)

        pltpu.make_async_copy(h_hbm.at[src_v.at[r]], rows_v.at[r],
                              gsems[r]).wait()

        @pl.when(j >= 1)
        def _():
            pltpu.make_async_copy(rows_v.at[r1], agg_sh.at[dst_v.at[d1]],
                                  ssems[p1]).wait()

        pltpu.make_async_copy(dst_hbm.at[wid, j], dst_v.at[d],
                              idst[d]).wait()

        pltpu.async_copy(rows_v.at[r], agg_sh.at[dst_v.at[d]], ssems[p],
                         add=True)

        @pl.when(j + 3 < CPW)
        def _():
            pltpu.async_copy(src_hbm.at[wid, j + 3], src_v.at[r], isrc[r])
            pltpu.async_copy(dst_hbm.at[wid, j + 3], dst_v.at[d1], idst[d1])

        @pl.when(j + 2 < CPW)
        def _():
            pltpu.make_async_copy(src_hbm.at[wid, j + 2], src_v.at[r1],
                                  isrc[r1]).wait()
            pltpu.async_copy(h_hbm.at[src_v.at[r1]], rows_v.at[r1], gsems[r1])

    def body(i, carry):
        for u in range(12):
            _step(12 * i + u, u)
        return carry

    lax.fori_loop(0, CPW // 12, body, 0)
    pltpu.make_async_copy(rows_v.at[(CPW - 1) % 3],
                          agg_sh.at[dst_v.at[(CPW - 1) % 4]],
                          ssems[(CPW - 1) % 2]).wait()
    plsc.subcore_barrier()

    pltpu.sync_copy(agg_sh.at[pl.ds(row0, ROWS_PER_TILE)],
                    out_hbm.at[cid, pl.ds(row0, ROWS_PER_TILE)])


ROW_BLK = 400


def _tc_mlp_body(h_ref, a0_ref, a1_ref, w1_ref, b1_ref, w2_ref, b2_ref, out_ref):
    m = h_ref[...] + a0_ref[0] + a1_ref[0]
    t = jnp.dot(m, w1_ref[...], preferred_element_type=jnp.float32) + b1_ref[...]
    t = jnp.maximum(t, 0.0)
    u = jnp.dot(t, w2_ref[...], preferred_element_type=jnp.float32) + b2_ref[...]
    out_ref[...] = jnp.where(u > 0.0, u, jnp.exp(jnp.minimum(u, 0.0)) - 1.0)


def _tc_mlp(h, agg, w1, b1, w2, b2):
    grid = N // ROW_BLK
    return pl.pallas_call(
        _tc_mlp_body,
        grid=(grid,),
        in_specs=[
            pl.BlockSpec((ROW_BLK, D), lambda i: (i, 0)),
            pl.BlockSpec((1, ROW_BLK, D), lambda i: (0, i, 0)),
            pl.BlockSpec((1, ROW_BLK, D), lambda i: (1, i, 0)),
            pl.BlockSpec((D, D), lambda i: (0, 0)),
            pl.BlockSpec((1, D), lambda i: (0, 0)),
            pl.BlockSpec((D, D), lambda i: (0, 0)),
            pl.BlockSpec((1, D), lambda i: (0, 0)),
        ],
        out_specs=pl.BlockSpec((ROW_BLK, D), lambda i: (i, 0)),
        out_shape=jax.ShapeDtypeStruct((N, D), jnp.float32),
    )(h, agg, agg, w1, b1, w2, b2)


def kernel(x, edge_index, W1_0, b1_0, W2_0, b2_0, W1_1, b1_1, W2_1, b2_1,
           W1_2, b1_2, W2_2, b2_2):
    src = edge_index[0]
    dst = edge_index[1]
    pad = E_PAD - E
    pad_iota = np.arange(pad, dtype=np.int32)
    pad_src = jnp.asarray(pad_iota % N)
    pad_dst = jnp.asarray(N + pad_iota % (N_SP - N))
    src_p = jnp.concatenate([src, pad_src]).reshape(NW, CPW, CHUNK)
    dst_p = jnp.concatenate([dst, pad_dst]).reshape(NW, CPW, CHUNK)
    zeros = jnp.zeros((N_SP, D), jnp.float32)

    params = [(W1_0, b1_0, W2_0, b2_0), (W1_1, b1_1, W2_1, b2_1),
              (W1_2, b1_2, W2_2, b2_2)]
    h = x
    for (w1, b1, w2, b2) in params:
        agg = _sc_agg(h, src_p, dst_p, zeros)
        h = _tc_mlp(h, agg, w1, b1.reshape(1, D), w2, b2.reshape(1, D))
    return h


# speedup vs baseline: 2.7262x; 1.0946x over previous
import functools

import jax
import jax.numpy as jnp
import numpy as np
from jax import lax
from jax.experimental import pallas as pl
from jax.experimental.pallas import tpu as pltpu
from jax.experimental.pallas import tpu_sc as plsc

N = 10000
E = 320000
D = 128

NC = 2
NS = 16
NW = NC * NS

CHUNK = 120
CPW = 84
EPW = CPW * CHUNK
E_PAD = EPW * NW

ROWS_PER_TILE = 632
N_SP = ROWS_PER_TILE * NS

_sc_mesh = plsc.VectorSubcoreMesh(core_axis_name="c", subcore_axis_name="s")


@functools.partial(
    pl.kernel,
    out_type=jax.ShapeDtypeStruct((NC, N_SP, D), jnp.float32),
    mesh=_sc_mesh,
    scratch_types=[
        pltpu.VMEM((3, CHUNK), jnp.int32),
        pltpu.VMEM((4, CHUNK), jnp.int32),
        pltpu.VMEM((3, CHUNK, D), jnp.float32),
        pltpu.VMEM_SHARED((N_SP, D), jnp.float32),
        [pltpu.SemaphoreType.DMA] * 3,
        [pltpu.SemaphoreType.DMA] * 2,
        [pltpu.SemaphoreType.DMA] * 3,
        [pltpu.SemaphoreType.DMA] * 4,
        pltpu.SemaphoreType.DMA,
    ],
)
def _sc_agg(h_hbm, src_hbm, dst_hbm, zeros_hbm, out_hbm,
            src_v, dst_v, rows_v, agg_sh, gsems, ssems, isrc, idst, zsem):
    cid = lax.axis_index("c")
    sid = lax.axis_index("s")
    wid = cid * NS + sid

    row0 = sid * ROWS_PER_TILE
    pltpu.async_copy(zeros_hbm.at[pl.ds(row0, ROWS_PER_TILE)],
                     agg_sh.at[pl.ds(row0, ROWS_PER_TILE)], zsem)
    for s in (0, 1, 2):
        pltpu.async_copy(src_hbm.at[wid, s], src_v.at[s], isrc[s])
        pltpu.async_copy(dst_hbm.at[wid, s], dst_v.at[s], idst[s])
    pltpu.make_async_copy(src_hbm.at[wid, 0], src_v.at[0], isrc[0]).wait()
    pltpu.async_copy(h_hbm.at[src_v.at[0]], rows_v.at[0], gsems[0])
    pltpu.make_async_copy(src_hbm.at[wid, 1], src_v.at[1], isrc[1]).wait()
    pltpu.async_copy(h_hbm.at[src_v.at[1]], rows_v.at[1], gsems[1])
    pltpu.make_async_copy(zeros_hbm.at[pl.ds(row0, ROWS_PER_TILE)],
                          agg_sh.at[pl.ds(row0, ROWS_PER_TILE)], zsem).wait()
    plsc.subcore_barrier()

    def _step(j, u):
        r, d, p = u % 3, u % 4, u % 2
        r1, d1, p1 = (u + 2) % 3, (u + 3) % 4, 1 - (u % 2)

        pltpu.make_async_copy(h_hbm.at[src_v.at[r]], rows_v.at[r],
                              gsems[r]).wait()

        @pl.when(j >= 1)
        def _():
            pltpu.make_async_copy(rows_v.at[r1], agg_sh.at[dst_v.at[d1]],
                                  ssems[p1]).wait()

        pltpu.make_async_copy(dst_hbm.at[wid, j], dst_v.at[d],
                              idst[d]).wait()

        pltpu.async_copy(rows_v.at[r], agg_sh.at[dst_v.at[d]], ssems[p],
                         add=True)

        @pl.when(j + 3 < CPW)
        def _():
            pltpu.async_copy(src_hbm.at[wid, j + 3], src_v.at[r], isrc[r])
            pltpu.async_copy(dst_hbm.at[wid, j + 3], dst_v.at[d1], idst[d1])

        @pl.when(j + 2 < CPW)
        def _():
            pltpu.make_async_copy(src_hbm.at[wid, j + 2], src_v.at[r1],
                                  isrc[r1]).wait()
            pltpu.async_copy(h_hbm.at[src_v.at[r1]], rows_v.at[r1], gsems[r1])

    def body(i, carry):
        for u in range(12):
            _step(12 * i + u, u)
        return carry

    lax.fori_loop(0, CPW // 12, body, 0)
    pltpu.make_async_copy(rows_v.at[(CPW - 1) % 3],
                          agg_sh.at[dst_v.at[(CPW - 1) % 4]],
                          ssems[(CPW - 1) % 2]).wait()
    plsc.subcore_barrier()

    pltpu.sync_copy(agg_sh.at[pl.ds(row0, ROWS_PER_TILE)],
                    out_hbm.at[cid, pl.ds(row0, ROWS_PER_TILE)])


ROW_BLK = 10000


def _tc_mlp_body(h_ref, a0_ref, a1_ref, w1_ref, b1_ref, w2_ref, b2_ref, out_ref):
    m = h_ref[...] + a0_ref[0] + a1_ref[0]
    t = jnp.dot(m, w1_ref[...], preferred_element_type=jnp.float32) + b1_ref[...]
    t = jnp.maximum(t, 0.0)
    u = jnp.dot(t, w2_ref[...], preferred_element_type=jnp.float32) + b2_ref[...]
    out_ref[...] = jnp.where(u > 0.0, u, jnp.exp(jnp.minimum(u, 0.0)) - 1.0)


def _tc_mlp(h, agg, w1, b1, w2, b2):
    grid = N // ROW_BLK
    return pl.pallas_call(
        _tc_mlp_body,
        grid=(grid,),
        in_specs=[
            pl.BlockSpec((ROW_BLK, D), lambda i: (i, 0)),
            pl.BlockSpec((1, ROW_BLK, D), lambda i: (0, i, 0)),
            pl.BlockSpec((1, ROW_BLK, D), lambda i: (1, i, 0)),
            pl.BlockSpec((D, D), lambda i: (0, 0)),
            pl.BlockSpec((1, D), lambda i: (0, 0)),
            pl.BlockSpec((D, D), lambda i: (0, 0)),
            pl.BlockSpec((1, D), lambda i: (0, 0)),
        ],
        out_specs=pl.BlockSpec((ROW_BLK, D), lambda i: (i, 0)),
        out_shape=jax.ShapeDtypeStruct((N, D), jnp.float32),
    )(h, agg, agg, w1, b1, w2, b2)


def kernel(x, edge_index, W1_0, b1_0, W2_0, b2_0, W1_1, b1_1, W2_1, b2_1,
           W1_2, b1_2, W2_2, b2_2):
    src = edge_index[0]
    dst = edge_index[1]
    pad = E_PAD - E
    pad_iota = np.arange(pad, dtype=np.int32)
    pad_src = jnp.asarray(pad_iota % N)
    pad_dst = jnp.asarray(N + pad_iota % (N_SP - N))
    src_p = jnp.concatenate([src, pad_src]).reshape(NW, CPW, CHUNK)
    dst_p = jnp.concatenate([dst, pad_dst]).reshape(NW, CPW, CHUNK)
    zeros = jnp.zeros((N_SP, D), jnp.float32)

    params = [(W1_0, b1_0, W2_0, b2_0), (W1_1, b1_1, W2_1, b2_1),
              (W1_2, b1_2, W2_2, b2_2)]
    h = x
    for (w1, b1, w2, b2) in params:
        agg = _sc_agg(h, src_p, dst_p, zeros)
        h = _tc_mlp(h, agg, w1, b1.reshape(1, D), w2, b2.reshape(1, D))
    return h
